# baseline (device time: 21792 ns/iter reference)
import jax
import jax.numpy as jnp
from jax import lax
from jax.experimental import pallas as pl
from jax.experimental.pallas import tpu as pltpu

N_DEV = 16
SIDE = 8


def kernel(x):
    m, n = x.shape

    def body(x_ref, out_ref, own_send, own_recv, rel_send, rel_recv,
             casc_send, casc_recv, pex_send, pex_recv, cim_send, cim_recv):
        my = lax.axis_index("i")
        z = my // 4
        j = my % 4
        side_base = (my // SIDE) * SIDE
        w = my % SIDE
        partner = 4 * (3 - z) + j
        cim = 4 * (z + 1 - 2 * (z % 2)) + j

        def partner_of(p):
            return 4 * (3 - p // 4) + p % 4

        def in_side_peer(t):
            return side_base + (w + t) % SIDE

        def in_side_src(t):
            return side_base + (w - t) % SIDE

        def plane_mate(u):
            return 4 * z + (j + u) % 4

        def plane_src(u):
            return 4 * z + (j - u) % 4

        def copy(src_slot, dst_slot, ssem, rsem, dev):
            return pltpu.make_async_remote_copy(
                src_ref=out_ref.at[src_slot],
                dst_ref=out_ref.at[dst_slot],
                send_sem=ssem,
                recv_sem=rsem,
                device_id=(dev,),
                device_id_type=pl.DeviceIdType.MESH,
            )

        barrier = pltpu.get_barrier_semaphore()
        for t in range(1, SIDE):
            pl.semaphore_signal(
                barrier, inc=1,
                device_id=(in_side_peer(t),),
                device_id_type=pl.DeviceIdType.MESH,
            )
        pl.semaphore_signal(
            barrier, inc=1,
            device_id=(partner,),
            device_id_type=pl.DeviceIdType.MESH,
        )
        pl.semaphore_wait(barrier, SIDE)

        out_ref[my] = x_ref[...].astype(jnp.bfloat16)

        pex = copy(my, my, pex_send.at[0], pex_recv.at[0], partner)
        pex.start()

        for t in range(1, SIDE):
            copy(my, my, own_send.at[t - 1], own_recv.at[t - 1],
                 in_side_peer(t)).start()

        pex.wait_recv()
        for u in range(1, 4):
            copy(partner, partner, rel_send.at[u - 1], rel_recv.at[u - 1],
                 plane_mate(u)).start()
        cimc = copy(partner, partner, cim_send.at[0], cim_recv.at[0], cim)
        cimc.start()

        copy(my, partner_of(cim), cim_send.at[0], cim_recv.at[0],
             cim).wait_recv()
        for u in range(1, 4):
            copy(partner_of(cim), partner_of(cim),
                 casc_send.at[u - 1], casc_recv.at[u - 1],
                 plane_mate(u)).start()

        for t in range(1, SIDE):
            copy(my, in_side_src(t), own_send.at[t - 1], own_recv.at[t - 1],
                 in_side_peer(t)).wait_recv()
        for u in range(1, 4):
            copy(my, partner_of(plane_src(u)),
                 rel_send.at[u - 1], rel_recv.at[u - 1],
                 plane_mate(u)).wait_recv()
        for u in range(1, 4):
            src = plane_src(u)
            chunk = partner_of(4 * (src // 4 + 1 - 2 * ((src // 4) % 2))
                               + src % 4)
            copy(my, chunk, casc_send.at[u - 1], casc_recv.at[u - 1],
                 plane_mate(u)).wait_recv()

        pex.wait_send()
        cimc.wait_send()
        for t in range(1, SIDE):
            copy(my, my, own_send.at[t - 1], own_recv.at[t - 1],
                 in_side_peer(t)).wait_send()
        for u in range(1, 4):
            copy(my, my, rel_send.at[u - 1], rel_recv.at[u - 1],
                 plane_mate(u)).wait_send()
            copy(my, my, casc_send.at[u - 1], casc_recv.at[u - 1],
                 plane_mate(u)).wait_send()

    out = pl.pallas_call(
        body,
        out_shape=jax.ShapeDtypeStruct((N_DEV, m, n), jnp.bfloat16),
        in_specs=[pl.BlockSpec(memory_space=pltpu.VMEM)],
        out_specs=pl.BlockSpec(memory_space=pltpu.VMEM),
        scratch_shapes=[
            pltpu.SemaphoreType.DMA((SIDE - 1,)),
            pltpu.SemaphoreType.DMA((SIDE - 1,)),
            pltpu.SemaphoreType.DMA((3,)),
            pltpu.SemaphoreType.DMA((3,)),
            pltpu.SemaphoreType.DMA((3,)),
            pltpu.SemaphoreType.DMA((3,)),
            pltpu.SemaphoreType.DMA((1,)),
            pltpu.SemaphoreType.DMA((1,)),
            pltpu.SemaphoreType.DMA((1,)),
            pltpu.SemaphoreType.DMA((1,)),
        ],
        compiler_params=pltpu.CompilerParams(collective_id=0),
    )(x)
    return out.reshape(N_DEV * m, n)


# device time: 21210 ns/iter; 1.0274x vs baseline; 1.0274x over previous
import jax
import jax.numpy as jnp
from jax import lax
from jax.experimental import pallas as pl
from jax.experimental.pallas import tpu as pltpu

N_DEV = 16
SIDE = 8


def kernel(x):
    m, n = x.shape

    def body(x_ref, out_ref, own_send, own_recv, rel_send, rel_recv,
             pex_send, pex_recv):
        my = lax.axis_index("i")
        side_base = (my // SIDE) * SIDE
        w = my % SIDE
        partner = 4 * (3 - my // 4) + my % 4

        def in_side_peer(t):
            return side_base + (w + t) % SIDE

        def in_side_src(t):
            return side_base + (w - t) % SIDE

        barrier = pltpu.get_barrier_semaphore()
        for t in range(1, SIDE):
            pl.semaphore_signal(
                barrier, inc=1,
                device_id=(in_side_peer(t),),
                device_id_type=pl.DeviceIdType.MESH,
            )
        pl.semaphore_signal(
            barrier, inc=1,
            device_id=(partner,),
            device_id_type=pl.DeviceIdType.MESH,
        )
        pl.semaphore_wait(barrier, SIDE)

        out_ref[my] = x_ref[...].astype(jnp.bfloat16)

        pex = pltpu.make_async_remote_copy(
            src_ref=out_ref.at[my],
            dst_ref=out_ref.at[my],
            send_sem=pex_send.at[0],
            recv_sem=pex_recv.at[0],
            device_id=(partner,),
            device_id_type=pl.DeviceIdType.MESH,
        )
        pex.start()

        for t in range(1, SIDE):
            rdma = pltpu.make_async_remote_copy(
                src_ref=out_ref.at[my],
                dst_ref=out_ref.at[my],
                send_sem=own_send.at[t - 1],
                recv_sem=own_recv.at[t - 1],
                device_id=(in_side_peer(t),),
                device_id_type=pl.DeviceIdType.MESH,
            )
            rdma.start()

        pex_w = pltpu.make_async_remote_copy(
            src_ref=out_ref.at[my],
            dst_ref=out_ref.at[partner],
            send_sem=pex_send.at[0],
            recv_sem=pex_recv.at[0],
            device_id=(partner,),
            device_id_type=pl.DeviceIdType.MESH,
        )
        pex_w.wait_recv()

        for t in range(1, SIDE):
            rdma = pltpu.make_async_remote_copy(
                src_ref=out_ref.at[partner],
                dst_ref=out_ref.at[partner],
                send_sem=rel_send.at[t - 1],
                recv_sem=rel_recv.at[t - 1],
                device_id=(in_side_peer(t),),
                device_id_type=pl.DeviceIdType.MESH,
            )
            rdma.start()

        for t in range(1, SIDE):
            src = in_side_src(t)
            recv = pltpu.make_async_remote_copy(
                src_ref=out_ref.at[my],
                dst_ref=out_ref.at[src],
                send_sem=own_send.at[t - 1],
                recv_sem=own_recv.at[t - 1],
                device_id=(in_side_peer(t),),
                device_id_type=pl.DeviceIdType.MESH,
            )
            recv.wait_recv()
        for t in range(1, SIDE):
            src = in_side_src(t)
            src_partner = 4 * (3 - src // 4) + src % 4
            recv = pltpu.make_async_remote_copy(
                src_ref=out_ref.at[my],
                dst_ref=out_ref.at[src_partner],
                send_sem=rel_send.at[t - 1],
                recv_sem=rel_recv.at[t - 1],
                device_id=(in_side_peer(t),),
                device_id_type=pl.DeviceIdType.MESH,
            )
            recv.wait_recv()

        pex_w.wait_send()
        for sems in (own_send, rel_send):
            for t in range(1, SIDE):
                send = pltpu.make_async_remote_copy(
                    src_ref=out_ref.at[my],
                    dst_ref=out_ref.at[my],
                    send_sem=sems.at[t - 1],
                    recv_sem=own_recv.at[t - 1],
                    device_id=(in_side_peer(t),),
                    device_id_type=pl.DeviceIdType.MESH,
                )
                send.wait_send()

    out = pl.pallas_call(
        body,
        out_shape=jax.ShapeDtypeStruct((N_DEV, m, n), jnp.bfloat16),
        in_specs=[pl.BlockSpec(memory_space=pltpu.VMEM)],
        out_specs=pl.BlockSpec(memory_space=pltpu.VMEM),
        scratch_shapes=[
            pltpu.SemaphoreType.DMA((SIDE - 1,)),
            pltpu.SemaphoreType.DMA((SIDE - 1,)),
            pltpu.SemaphoreType.DMA((SIDE - 1,)),
            pltpu.SemaphoreType.DMA((SIDE - 1,)),
            pltpu.SemaphoreType.DMA((1,)),
            pltpu.SemaphoreType.DMA((1,)),
        ],
        compiler_params=pltpu.CompilerParams(collective_id=0),
    )(x)
    return out.reshape(N_DEV * m, n)
